# baseline (speedup 1.0000x reference)
import jax
import jax.numpy as jnp
from jax import lax
from jax.experimental import pallas as pl
from jax.experimental.pallas import tpu as pltpu
from jax.experimental.pallas import tpu_sc as plsc

_D = 4
_L = 6
_H = 128
_W = 128
_NCLS = 10
_NT = 16
_RPT = 5
_SLAB0 = 48
_OUTR = 64
_NCG = _W // 16
_DIRS = ((-1, 0), (1, 0), (0, -1), (0, 1))

_f32 = jnp.float32
_i32 = jnp.int32

_O_NW = 0
_O_NB = 4
_O_EW = 8
_O_EB = 24
_O_M1 = 28
_O_MB1 = _O_M1 + _L * 48
_O_M2 = _O_MB1 + _L * 4
_O_MB2 = _O_M2 + _L * 16
_O_U1 = _O_MB2 + _L * 4
_O_UB1 = _O_U1 + _L * 32
_O_U2 = _O_UB1 + _L * 4
_O_UB2 = _O_U2 + _L * 16
_O_OW = _O_UB2 + _L * 4
_O_OB = _O_OW + _NCLS * 4
_NVEC = _O_OB + _NCLS
_WROWS = ((_NVEC + 7) // 8 + 7) // 8 * 8


def _sc_body(grid_h, wall_h, prow_h, jtab_h, out_h,
             xa, xb, wall, gbuf, obuf, sh, prowb, jtabb):
    t = lax.axis_index("s")
    c01 = lax.axis_index("c")
    base = c01 * _SLAB0
    z16 = jnp.zeros((16,), _f32)
    one16 = jnp.full((16,), 1.0, _f32)
    iota16 = lax.iota(_i32, 16)
    rmax16 = jnp.full((16,), _H - 2.0, _f32)
    rmin16 = jnp.full((16,), 1.0, _f32)
    cmax16 = jnp.full((16,), _W - 2.0, _f32)
    cmin16 = jnp.full((16,), 1.0, _f32)
    idxm1 = jnp.maximum(iota16 - 1, 0)
    idxp1 = jnp.minimum(iota16 + 1, 15)
    idx0v = jnp.zeros((16,), _i32)
    idx15v = jnp.full((16,), 15, _i32)
    eq0 = iota16 == idx0v
    eq15 = iota16 == idx15v
    _gdn = lax.GatherDimensionNumbers(offset_dims=(),
                                      collapsed_slice_dims=(0,),
                                      start_index_map=(0,))

    def _dg(v, idx):
        return lax.gather(v, idx[:, None], _gdn, (1,),
                          mode=lax.GatherScatterMode.PROMISE_IN_BOUNDS)

    def _shift_r(prev, cur):
        return jnp.where(eq0, _dg(prev, idx15v), _dg(cur, idxm1))

    def _shift_l(cur, nxt):
        return jnp.where(eq15, _dg(nxt, idx0v), _dg(cur, idxp1))

    def W(v):
        return wall[v // 8, pl.ds((v % 8) * 16, 16)]

    pltpu.sync_copy(wall_h, wall)
    for rr in range(_RPT):
        pltpu.sync_copy(grid_h.at[base + t * _RPT + rr], gbuf.at[rr])
    pltpu.sync_copy(prow_h.at[c01 * _NT + t], prowb.at[0])
    pltpu.sync_copy(jtab_h, jtabb)
    for buf in (xa, xb):
        for c in range(_D):
            for b in range(_NCG):
                buf[c, pl.ds(b * 16, 16)] = z16
                buf[(_RPT + 1) * _D + c, pl.ds(b * 16, 16)] = z16

    def _xinit(idx, carry):
        r = (idx >> 3) + 1
        col0 = (idx & 7) * 16
        g = gbuf[r - 1, pl.ds(col0, 16)].astype(_f32)
        for c in range(_D):
            xa[r * _D + c, pl.ds(col0, 16)] = g * W(_O_NW + c) + W(_O_NB + c)
        return carry

    lax.fori_loop(0, _RPT * _NCG, _xinit, 0)

    def _shrow(parity, tt, side, c):
        return ((parity * _NT + tt) * 2 + side) * _D + c

    def _exchange(buf, parity):
        pltpu.sync_copy(buf.at[pl.ds(1 * _D, _D), :],
                        sh.at[pl.ds(_shrow(parity, t, 0, 0), _D)])
        pltpu.sync_copy(buf.at[pl.ds(_RPT * _D, _D), :],
                        sh.at[pl.ds(_shrow(parity, t, 1, 0), _D)])
        plsc.subcore_barrier()

        @pl.when(t > 0)
        def _():
            pltpu.sync_copy(sh.at[pl.ds(_shrow(parity, t - 1, 1, 0), _D)],
                            buf.at[pl.ds(0, _D), :])

        @pl.when(t < _NT - 1)
        def _():
            pltpu.sync_copy(sh.at[pl.ds(_shrow(parity, t + 1, 0, 0), _D)],
                            buf.at[pl.ds((_RPT + 1) * _D, _D), :])

    _exchange(xa, 0)

    for l in range(_L):
        src, dst = (xa, xb) if l % 2 == 0 else (xb, xa)
        ew = [[W(_O_EW + c * 4 + k) for k in range(4)] for c in range(_D)]
        m1 = [[W(_O_M1 + l * 48 + h * 12 + k) for k in range(12)]
              for h in range(_D)]
        beta, gamma, eta = [], [], []
        for h in range(_D):
            b = m1[h][8] * ew[0][2]
            g = m1[h][8] * ew[0][3]
            e = m1[h][8] * W(_O_EB + 0)
            for c in range(1, _D):
                b = b + m1[h][8 + c] * ew[c][2]
                g = g + m1[h][8 + c] * ew[c][3]
                e = e + m1[h][8 + c] * W(_O_EB + c)
            beta.append(b * (1.0 / _H))
            gamma.append(g * (1.0 / _W))
            eta.append(e + W(_O_MB1 + l * 4 + h))
        da = [ew[c][0] - ew[c][2] * (1.0 / _H) for c in range(_D)]
        db = [ew[c][1] - ew[c][3] * (1.0 / _W) for c in range(_D)]
        alpha = []
        for (di, dj) in _DIRS:
            ah = []
            for h in range(_D):
                acc = None
                for c in range(_D):
                    if di:
                        term = da[c] if di > 0 else -da[c]
                    else:
                        term = db[c] if dj > 0 else -db[c]
                    v = m1[h][8 + c] * term
                    acc = v if acc is None else acc + v
                ah.append(acc)
            alpha.append(ah)

        def _node(idx, carry, l=l, src=src, dst=dst, beta=beta, gamma=gamma,
                  eta=eta, alpha=alpha, m1=m1):
            r = (idx >> 2) + 1
            cgp = idx & 3
            gi_vec = prowb[0, pl.ds((r - 1) * 16, 16)]
            mrow_lo = jnp.where(gi_vec <= rmax16, one16, z16)
            mrow_hi = jnp.where(gi_vec >= rmin16, one16, z16)
            m2v = [[W(_O_M2 + l * 16 + c * 4 + h) for h in range(_D)]
                   for c in range(_D)]
            mb2v = [W(_O_MB2 + l * 4 + c) for c in range(_D)]
            u1v = [[W(_O_U1 + l * 32 + h * 8 + k) for k in range(2 * _D)]
                   for h in range(_D)]
            ub1v = [W(_O_UB1 + l * 4 + h) for h in range(_D)]
            u2v = [[W(_O_U2 + l * 16 + c * 4 + h) for h in range(_D)]
                   for c in range(_D)]
            ub2v = [W(_O_UB2 + l * 4 + c) for c in range(_D)]
            for sub in range(2):
                col0 = cgp * 32 + sub * 16
                jidx = jtabb[0, pl.ds(col0, 16)]
                xc = [src[r * _D + c, pl.ds(col0, 16)] for c in range(_D)]
                tsh = []
                for h in range(_D):
                    acc = xc[0] * m1[h][_D]
                    for c in range(1, _D):
                        acc = acc + xc[c] * m1[h][_D + c]
                    acc = acc + beta[h] * gi_vec + gamma[h] * jidx + eta[h]
                    tsh.append(acc)
                masks = [
                    mrow_lo,
                    mrow_hi,
                    jnp.where(jidx <= cmax16, one16, z16),
                    jnp.where(jidx >= cmin16, one16, z16),
                ]
                agg = [z16, z16, z16, z16]
                for d, (di, dj) in enumerate(_DIRS):
                    if dj == 0:
                        sf = [src[(r - di) * _D + c, pl.ds(col0, 16)]
                              for c in range(_D)]
                    elif dj > 0:
                        cp = jnp.maximum(col0 - 16, 0)
                        sf = [_shift_r(src[r * _D + c, pl.ds(cp, 16)], xc[c])
                              for c in range(_D)]
                    else:
                        cn = jnp.minimum(col0 + 16, _W - 16)
                        sf = [_shift_l(xc[c], src[r * _D + c, pl.ds(cn, 16)])
                              for c in range(_D)]
                    hid = []
                    for h in range(_D):
                        acc = tsh[h] + alpha[d][h]
                        for c in range(_D):
                            acc = acc + sf[c] * m1[h][c]
                        hid.append(jnp.maximum(acc, z16))
                    for c in range(_D):
                        m = hid[0] * m2v[c][0]
                        for h in range(1, _D):
                            m = m + hid[h] * m2v[c][h]
                        agg[c] = agg[c] + masks[d] * (m + mb2v[c])
                hid2 = []
                for h in range(_D):
                    acc = xc[0] * u1v[h][0]
                    for c in range(1, _D):
                        acc = acc + xc[c] * u1v[h][c]
                    for c in range(_D):
                        acc = acc + agg[c] * u1v[h][_D + c]
                    hid2.append(jnp.maximum(acc + ub1v[h], z16))
                for c in range(_D):
                    acc = hid2[0] * u2v[c][0]
                    for h in range(1, _D):
                        acc = acc + hid2[h] * u2v[c][h]
                    dst[r * _D + c, pl.ds(col0, 16)] = acc + ub2v[c]
            return carry

        lax.fori_loop(0, _RPT * _NCG // 2, _node, 0)
        if l < _L - 1:
            _exchange(dst, (l + 1) % 2)

    def _head(r, carry):
        for b in range(_NCG):
            col0 = b * 16
            xc = [xa[r * _D + c, pl.ds(col0, 16)] for c in range(_D)]
            for k in range(_NCLS):
                acc = xc[0] * W(_O_OW + k * 4 + 0)
                for c in range(1, _D):
                    acc = acc + xc[c] * W(_O_OW + k * 4 + c)
                obuf[k, r - 1, pl.ds(col0, 16)] = acc + W(_O_OB + k)
        return carry

    lax.fori_loop(1, _RPT + 1, _head, 0)
    pltpu.sync_copy(obuf, out_h.at[c01, t])


def kernel(grid, node_w, node_b, edge_w, edge_b, msg_w1, msg_b1, msg_w2,
           msg_b2, upd_w1, upd_b1, upd_w2, upd_b2, out_w, out_b):
    mesh = plsc.VectorSubcoreMesh(core_axis_name="c", subcore_axis_name="s",
                                  num_cores=2, num_subcores=_NT)
    fn = pl.kernel(
        _sc_body,
        out_type=jax.ShapeDtypeStruct((2, _NT, _NCLS, _RPT, _W), _f32),
        mesh=mesh,
        scratch_types=[
            pltpu.VMEM((_D * (_RPT + 2), _W), _f32),
            pltpu.VMEM((_D * (_RPT + 2), _W), _f32),
            pltpu.VMEM((_WROWS, 128), _f32),
            pltpu.VMEM((_RPT, _W), _i32),
            pltpu.VMEM((_NCLS, _RPT, _W), _f32),
            pltpu.VMEM_SHARED((2 * _NT * 2 * _D, _W), _f32),
            pltpu.VMEM((1, _RPT * 16), _f32),
            pltpu.VMEM((1, 128), _f32),
        ],
    )
    scalars = jnp.concatenate([
        node_w[:, 0], node_b, edge_w.reshape(-1), edge_b,
        msg_w1.reshape(-1), msg_b1.reshape(-1), msg_w2.reshape(-1),
        msg_b2.reshape(-1), upd_w1.reshape(-1), upd_b1.reshape(-1),
        upd_w2.reshape(-1), upd_b2.reshape(-1), out_w.reshape(-1), out_b,
    ]).astype(_f32)
    scalars = jnp.pad(scalars, (0, _WROWS * 8 - _NVEC))
    wall = jnp.broadcast_to(scalars.reshape(_WROWS, 8)[..., None],
                            (_WROWS, 8, 16)).reshape(_WROWS, 128)
    rows = (jnp.arange(2, dtype=_f32) * _SLAB0)[:, None] + jnp.arange(
        _NT * _RPT, dtype=_f32)[None, :]
    prow = jnp.repeat(rows.reshape(2 * _NT, _RPT), 16, axis=1)
    jtab = jnp.arange(_W, dtype=_f32).reshape(1, _W)
    out = fn(grid, wall, prow, jtab)
    flat = out.transpose(0, 1, 3, 2, 4).reshape(2 * _NT * _RPT, _NCLS, _W)
    gi = jnp.arange(_H)
    rows = flat[jnp.where(gi < _OUTR, gi, gi + 2 * _SLAB0 - _OUTR)]
    return jnp.transpose(rows, (0, 2, 1))


# speedup vs baseline: 1.1197x; 1.1197x over previous
import jax
import jax.numpy as jnp
from jax import lax
from jax.experimental import pallas as pl
from jax.experimental.pallas import tpu as pltpu
from jax.experimental.pallas import tpu_sc as plsc

_D = 4
_L = 6
_H = 128
_W = 128
_NCLS = 10
_NT = 16
_RPT = 5
_SLAB0 = 48
_OUTR = 64
_NCG = _W // 16
_DIRS = ((-1, 0), (1, 0), (0, -1), (0, 1))

_f32 = jnp.float32
_i32 = jnp.int32

_O_NW = 0
_O_NB = 4
_O_EW = 8
_O_EB = 24
_O_M1 = 28
_O_MB1 = _O_M1 + _L * 48
_O_M2 = _O_MB1 + _L * 4
_O_MB2 = _O_M2 + _L * 16
_O_U1 = _O_MB2 + _L * 4
_O_UB1 = _O_U1 + _L * 32
_O_U2 = _O_UB1 + _L * 4
_O_UB2 = _O_U2 + _L * 16
_O_OW = _O_UB2 + _L * 4
_O_OB = _O_OW + _NCLS * 4
_NVEC = _O_OB + _NCLS
_WROWS = ((_NVEC + 7) // 8 + 7) // 8 * 8


def _sc_body(grid_h, wall_h, prow_h, jtab_h, out_h,
             xa, xb, wall, gbuf, obuf, sh, prowb, jtabb):
    t = lax.axis_index("s")
    c01 = lax.axis_index("c")
    base = c01 * _SLAB0
    z16 = jnp.zeros((16,), _f32)
    one16 = jnp.full((16,), 1.0, _f32)
    iota16 = lax.iota(_i32, 16)
    rmax16 = jnp.full((16,), _H - 2.0, _f32)
    rmin16 = jnp.full((16,), 1.0, _f32)
    cmax16 = jnp.full((16,), _W - 2.0, _f32)
    cmin16 = jnp.full((16,), 1.0, _f32)
    idxm1 = jnp.maximum(iota16 - 1, 0)
    idxp1 = jnp.minimum(iota16 + 1, 15)
    idx0v = jnp.zeros((16,), _i32)
    idx15v = jnp.full((16,), 15, _i32)
    eq0 = iota16 == idx0v
    eq15 = iota16 == idx15v
    _gdn = lax.GatherDimensionNumbers(offset_dims=(),
                                      collapsed_slice_dims=(0,),
                                      start_index_map=(0,))

    def _dg(v, idx):
        return lax.gather(v, idx[:, None], _gdn, (1,),
                          mode=lax.GatherScatterMode.PROMISE_IN_BOUNDS)

    def _shift_r(prev, cur):
        return jnp.where(eq0, _dg(prev, idx15v), _dg(cur, idxm1))

    def _shift_l(cur, nxt):
        return jnp.where(eq15, _dg(nxt, idx0v), _dg(cur, idxp1))

    def W(v):
        return wall[v // 8, pl.ds((v % 8) * 16, 16)]

    pltpu.sync_copy(wall_h, wall)
    for rr in range(_RPT):
        pltpu.sync_copy(grid_h.at[base + t * _RPT + rr], gbuf.at[rr])
    pltpu.sync_copy(prow_h.at[c01 * _NT + t], prowb.at[0])
    pltpu.sync_copy(jtab_h, jtabb)
    for buf in (xa, xb):
        for c in range(_D):
            for b in range(_NCG):
                buf[c, pl.ds(b * 16, 16)] = z16
                buf[(_RPT + 1) * _D + c, pl.ds(b * 16, 16)] = z16

    def _xinit(idx, carry):
        r = (idx >> 3) + 1
        col0 = (idx & 7) * 16
        g = gbuf[r - 1, pl.ds(col0, 16)].astype(_f32)
        for c in range(_D):
            xa[r * _D + c, pl.ds(col0, 16)] = g * W(_O_NW + c) + W(_O_NB + c)
        return carry

    lax.fori_loop(0, _RPT * _NCG, _xinit, 0)

    def _shrow(parity, tt, side, c):
        return ((parity * _NT + tt) * 2 + side) * _D + c

    def _exchange(buf, parity):
        pltpu.sync_copy(buf.at[pl.ds(1 * _D, _D), :],
                        sh.at[pl.ds(_shrow(parity, t, 0, 0), _D)])
        pltpu.sync_copy(buf.at[pl.ds(_RPT * _D, _D), :],
                        sh.at[pl.ds(_shrow(parity, t, 1, 0), _D)])
        plsc.subcore_barrier()

        @pl.when(t > 0)
        def _():
            pltpu.sync_copy(sh.at[pl.ds(_shrow(parity, t - 1, 1, 0), _D)],
                            buf.at[pl.ds(0, _D), :])

        @pl.when(t < _NT - 1)
        def _():
            pltpu.sync_copy(sh.at[pl.ds(_shrow(parity, t + 1, 0, 0), _D)],
                            buf.at[pl.ds((_RPT + 1) * _D, _D), :])

    _exchange(xa, 0)

    for l in range(_L):
        src, dst = (xa, xb) if l % 2 == 0 else (xb, xa)
        ew = [[W(_O_EW + c * 4 + k) for k in range(4)] for c in range(_D)]
        m1 = [[W(_O_M1 + l * 48 + h * 12 + k) for k in range(12)]
              for h in range(_D)]
        beta, gamma, eta = [], [], []
        for h in range(_D):
            b = m1[h][8] * ew[0][2]
            g = m1[h][8] * ew[0][3]
            e = m1[h][8] * W(_O_EB + 0)
            for c in range(1, _D):
                b = b + m1[h][8 + c] * ew[c][2]
                g = g + m1[h][8 + c] * ew[c][3]
                e = e + m1[h][8 + c] * W(_O_EB + c)
            beta.append(b * (1.0 / _H))
            gamma.append(g * (1.0 / _W))
            eta.append(e + W(_O_MB1 + l * 4 + h))
        da = [ew[c][0] - ew[c][2] * (1.0 / _H) for c in range(_D)]
        db = [ew[c][1] - ew[c][3] * (1.0 / _W) for c in range(_D)]
        alpha = []
        for (di, dj) in _DIRS:
            ah = []
            for h in range(_D):
                acc = None
                for c in range(_D):
                    if di:
                        term = da[c] if di > 0 else -da[c]
                    else:
                        term = db[c] if dj > 0 else -db[c]
                    v = m1[h][8 + c] * term
                    acc = v if acc is None else acc + v
                ah.append(acc)
            alpha.append(ah)

        def _node(idx, carry, l=l, src=src, dst=dst, beta=beta, gamma=gamma,
                  eta=eta, alpha=alpha, m1=m1):
            r = (idx >> 2) + 1
            cgp = idx & 3
            gi_vec = prowb[0, pl.ds((r - 1) * 16, 16)]
            mrow_lo = jnp.where(gi_vec <= rmax16, one16, z16)
            mrow_hi = jnp.where(gi_vec >= rmin16, one16, z16)
            m2v = [[W(_O_M2 + l * 16 + c * 4 + h) for h in range(_D)]
                   for c in range(_D)]
            mb2v = [W(_O_MB2 + l * 4 + c) for c in range(_D)]
            u1v = [[W(_O_U1 + l * 32 + h * 8 + k) for k in range(2 * _D)]
                   for h in range(_D)]
            ub1v = [W(_O_UB1 + l * 4 + h) for h in range(_D)]
            u2v = [[W(_O_U2 + l * 16 + c * 4 + h) for h in range(_D)]
                   for c in range(_D)]
            ub2v = [W(_O_UB2 + l * 4 + c) for c in range(_D)]
            for sub in range(2):
                col0 = cgp * 32 + sub * 16
                jidx = jtabb[0, pl.ds(col0, 16)]
                xc = [src[r * _D + c, pl.ds(col0, 16)] for c in range(_D)]
                tsh = []
                for h in range(_D):
                    acc = xc[0] * m1[h][_D]
                    for c in range(1, _D):
                        acc = acc + xc[c] * m1[h][_D + c]
                    acc = acc + beta[h] * gi_vec + gamma[h] * jidx + eta[h]
                    tsh.append(acc)
                masks = [
                    mrow_lo,
                    mrow_hi,
                    jnp.where(jidx <= cmax16, one16, z16),
                    jnp.where(jidx >= cmin16, one16, z16),
                ]
                agg = [z16, z16, z16, z16]
                for d, (di, dj) in enumerate(_DIRS):
                    if dj == 0:
                        sf = [src[(r - di) * _D + c, pl.ds(col0, 16)]
                              for c in range(_D)]
                    elif dj > 0:
                        cp = jnp.maximum(col0 - 16, 0)
                        sf = [_shift_r(src[r * _D + c, pl.ds(cp, 16)], xc[c])
                              for c in range(_D)]
                    else:
                        cn = jnp.minimum(col0 + 16, _W - 16)
                        sf = [_shift_l(xc[c], src[r * _D + c, pl.ds(cn, 16)])
                              for c in range(_D)]
                    hid = []
                    for h in range(_D):
                        acc = tsh[h] + alpha[d][h]
                        for c in range(_D):
                            acc = acc + sf[c] * m1[h][c]
                        hid.append(jnp.maximum(acc, z16))
                    for c in range(_D):
                        m = hid[0] * m2v[c][0]
                        for h in range(1, _D):
                            m = m + hid[h] * m2v[c][h]
                        agg[c] = agg[c] + masks[d] * (m + mb2v[c])
                hid2 = []
                for h in range(_D):
                    acc = xc[0] * u1v[h][0]
                    for c in range(1, _D):
                        acc = acc + xc[c] * u1v[h][c]
                    for c in range(_D):
                        acc = acc + agg[c] * u1v[h][_D + c]
                    hid2.append(jnp.maximum(acc + ub1v[h], z16))
                for c in range(_D):
                    acc = hid2[0] * u2v[c][0]
                    for h in range(1, _D):
                        acc = acc + hid2[h] * u2v[c][h]
                    dst[r * _D + c, pl.ds(col0, 16)] = acc + ub2v[c]
            return carry

        lax.fori_loop(0, _RPT * _NCG // 2, _node, 0)
        if l < _L - 1:
            _exchange(dst, (l + 1) % 2)

    def _head(r, carry):
        for b in range(_NCG):
            col0 = b * 16
            xc = [xa[r * _D + c, pl.ds(col0, 16)] for c in range(_D)]
            for k in range(_NCLS):
                acc = xc[0] * W(_O_OW + k * 4 + 0)
                for c in range(1, _D):
                    acc = acc + xc[c] * W(_O_OW + k * 4 + c)
                obuf[k, pl.ds(col0, 16)] = acc + W(_O_OB + k)
        gi = base + t * _RPT + r - 1
        lo = c01 * _OUTR

        @pl.when((gi >= lo) & (gi < lo + _OUTR))
        def _():
            pltpu.sync_copy(obuf, out_h.at[:, gi])
        return carry

    lax.fori_loop(1, _RPT + 1, _head, 0)


def kernel(grid, node_w, node_b, edge_w, edge_b, msg_w1, msg_b1, msg_w2,
           msg_b2, upd_w1, upd_b1, upd_w2, upd_b2, out_w, out_b):
    mesh = plsc.VectorSubcoreMesh(core_axis_name="c", subcore_axis_name="s",
                                  num_cores=2, num_subcores=_NT)
    fn = pl.kernel(
        _sc_body,
        out_type=jax.ShapeDtypeStruct((_NCLS, _H, _W), _f32),
        mesh=mesh,
        scratch_types=[
            pltpu.VMEM((_D * (_RPT + 2), _W), _f32),
            pltpu.VMEM((_D * (_RPT + 2), _W), _f32),
            pltpu.VMEM((_WROWS, 128), _f32),
            pltpu.VMEM((_RPT, _W), _i32),
            pltpu.VMEM((_NCLS, _W), _f32),
            pltpu.VMEM_SHARED((2 * _NT * 2 * _D, _W), _f32),
            pltpu.VMEM((1, _RPT * 16), _f32),
            pltpu.VMEM((1, 128), _f32),
        ],
    )
    scalars = jnp.concatenate([
        node_w[:, 0], node_b, edge_w.reshape(-1), edge_b,
        msg_w1.reshape(-1), msg_b1.reshape(-1), msg_w2.reshape(-1),
        msg_b2.reshape(-1), upd_w1.reshape(-1), upd_b1.reshape(-1),
        upd_w2.reshape(-1), upd_b2.reshape(-1), out_w.reshape(-1), out_b,
    ]).astype(_f32)
    scalars = jnp.pad(scalars, (0, _WROWS * 8 - _NVEC))
    wall = jnp.broadcast_to(scalars.reshape(_WROWS, 8)[..., None],
                            (_WROWS, 8, 16)).reshape(_WROWS, 128)
    rows = (jnp.arange(2, dtype=_f32) * _SLAB0)[:, None] + jnp.arange(
        _NT * _RPT, dtype=_f32)[None, :]
    prow = jnp.repeat(rows.reshape(2 * _NT, _RPT), 16, axis=1)
    jtab = jnp.arange(_W, dtype=_f32).reshape(1, _W)
    out = fn(grid, wall, prow, jtab)
    return jnp.transpose(out, (1, 2, 0))
